# baseline (speedup 1.0000x reference)
import functools

import jax
import jax.numpy as jnp
from jax import lax
from jax.experimental import pallas as pl
from jax.experimental.pallas import tpu as pltpu
from jax.experimental.pallas import tpu_sc as plsc

KNN = 20
KPAD = 32
ROW_T = 256
NEG_INF = float("-inf")


def _tc_body(xt_ref, x_ref, w1_ref, wz_ref, idx_ref, yz_ref, work_ref, *,
             n_full):
    b = pl.program_id(0)
    xt_t = xt_ref[0]
    xf = x_ref[0]

    dot = lax.dot_general(xt_t, xf, (((1,), (0,)), ((), ())),
                          preferred_element_type=jnp.float32)
    rn = jnp.sum(xt_t * xt_t, axis=1, keepdims=True)
    cn = jnp.sum(xf * xf, axis=0, keepdims=True)
    work0 = 2.0 * dot - rn - cn
    work_ref[...] = work0
    m0 = jnp.max(work0, axis=1, keepdims=True)

    t_rows = xt_t.shape[0]
    iota_f = lax.convert_element_type(
        lax.broadcasted_iota(jnp.int32, (t_rows, n_full), 1), jnp.float32)
    slot = lax.broadcasted_iota(jnp.int32, (t_rows, KPAD), 1)

    def t_body(t, carry):
        idx_acc, m = carry
        w = work_ref[...]
        cand = jnp.where(w == m, iota_f, float(n_full))
        j_f = jnp.min(cand, axis=1, keepdims=True)
        wn = jnp.where(iota_f == j_f, NEG_INF, w)
        work_ref[...] = wn
        m_next = jnp.max(wn, axis=1, keepdims=True)
        j = lax.convert_element_type(j_f, jnp.int32)
        return (jnp.where(slot == t, j, idx_acc), m_next)

    idx_acc, _ = lax.fori_loop(
        0, KNN, t_body, (jnp.zeros((t_rows, KPAD), jnp.int32), m0))
    idx_ref[0] = idx_acc + b * n_full

    y_t = jnp.dot(xt_t, w1_ref[...], preferred_element_type=jnp.float32)
    z_t = jnp.dot(xt_t, wz_ref[...], preferred_element_type=jnp.float32)
    yz_ref[0] = jnp.concatenate([y_t, z_t], axis=1)


def _tc_topk_proj(xt, x, w1t, wzt):
    B, N, C = xt.shape
    O = w1t.shape[1]
    nt = N // ROW_T
    return pl.pallas_call(
        functools.partial(_tc_body, n_full=N),
        grid=(B, nt),
        in_specs=[
            pl.BlockSpec((1, ROW_T, C), lambda b, i: (b, i, 0)),
            pl.BlockSpec((1, C, N), lambda b, i: (b, 0, 0)),
            pl.BlockSpec((C, O), lambda b, i: (0, 0)),
            pl.BlockSpec((C, O), lambda b, i: (0, 0)),
        ],
        out_specs=[
            pl.BlockSpec((1, ROW_T, KPAD), lambda b, i: (b, i, 0)),
            pl.BlockSpec((1, ROW_T, 2 * O), lambda b, i: (b, i, 0)),
        ],
        out_shape=[
            jax.ShapeDtypeStruct((B, N, KPAD), jnp.int32),
            jax.ShapeDtypeStruct((B, N, 2 * O), jnp.float32),
        ],
        scratch_shapes=[pltpu.VMEM((ROW_T, N), jnp.float32)],
    )(xt, x, w1t, wzt)


def _sc_gather_combine(yz2, idxf, O):
    R = yz2.shape[0]
    info = plsc.get_sparse_core_info()
    nw = info.num_cores * info.num_subcores
    rpw = R // nw
    chunk = 8
    half = chunk * KNN // 2
    nch = rpw // chunk
    nvec = O // 16

    mesh = plsc.VectorSubcoreMesh(core_axis_name="c", subcore_axis_name="s")

    @functools.partial(
        pl.kernel,
        out_type=jax.ShapeDtypeStruct((R, O), jnp.float32),
        mesh=mesh,
        scratch_types=[
            pltpu.VMEM((rpw * KNN,), jnp.int32),
            pltpu.VMEM((2, chunk * KNN, 2 * O), jnp.float32),
            pltpu.VMEM((2, chunk, 2 * O), jnp.float32),
            pltpu.VMEM((2, chunk, O), jnp.float32),
            pltpu.SemaphoreType.DMA,
            pltpu.SemaphoreType.DMA,
            pltpu.SemaphoreType.DMA,
        ],
    )
    def sc_kernel(yz_hbm, idx_hbm, out_hbm, idx_all, rows_v, z_v, acc_v,
                  gsem0, gsem1, osem):
        wid = lax.axis_index("s") * info.num_cores + lax.axis_index("c")
        base = wid * rpw
        gsems = [gsem0, gsem1]

        pltpu.sync_copy(idx_hbm.at[pl.ds(base * KNN, rpw * KNN)], idx_all)

        def issue(ci, buf):
            r0 = base + ci * chunk
            o = ci * chunk * KNN
            pltpu.async_copy(yz_hbm.at[idx_all.at[pl.ds(o, half)]],
                             rows_v.at[buf, pl.ds(0, half)], gsems[buf])
            pltpu.async_copy(yz_hbm.at[idx_all.at[pl.ds(o + half, half)]],
                             rows_v.at[buf, pl.ds(half, half)], gsems[buf])
            pltpu.async_copy(yz_hbm.at[pl.ds(r0, chunk)], z_v.at[buf],
                             gsems[buf])

        def drain(buf):
            pltpu.make_async_copy(yz_hbm.at[idx_all.at[pl.ds(0, half)]],
                                  rows_v.at[buf, pl.ds(0, half)],
                                  gsems[buf]).wait()
            pltpu.make_async_copy(yz_hbm.at[idx_all.at[pl.ds(0, half)]],
                                  rows_v.at[buf, pl.ds(half, half)],
                                  gsems[buf]).wait()
            pltpu.make_async_copy(yz_hbm.at[pl.ds(base, chunk)], z_v.at[buf],
                                  gsems[buf]).wait()

        def out_drain(buf):
            pltpu.make_async_copy(acc_v.at[buf],
                                  out_hbm.at[pl.ds(base, chunk)], osem).wait()

        def compute(ci, buf):
            r0 = base + ci * chunk
            for r in range(chunk):
                zv = [z_v[buf, r, pl.ds(O + v * 16, 16)] for v in range(nvec)]

                def k_body(k, accs, r=r, zv=zv):
                    out = []
                    for v in range(nvec):
                        g = rows_v[buf, r * KNN + k, pl.ds(v * 16, 16)]
                        s = g + zv[v]
                        out.append(accs[v] + (s - 0.8 * jnp.minimum(s, 0.0)))
                    return tuple(out)

                zero = jnp.zeros((16,), jnp.float32)
                accs = lax.fori_loop(0, KNN, k_body,
                                     tuple(zero for _ in range(nvec)))
                for v in range(nvec):
                    acc_v[buf, r, pl.ds(v * 16, 16)] = accs[v]
            pltpu.async_copy(acc_v.at[buf], out_hbm.at[pl.ds(r0, chunk)],
                             osem)

        issue(0, 0)

        def pair_body(g, _):
            c0 = 2 * g
            issue(c0 + 1, 1)
            drain(0)

            @pl.when(g > 0)
            def _():
                out_drain(0)
                out_drain(1)

            compute(c0, 0)

            @pl.when(g < nch // 2 - 1)
            def _():
                issue(c0 + 2, 0)

            drain(1)
            compute(c0 + 1, 1)
            return 0

        lax.fori_loop(0, nch // 2, pair_body, 0)
        out_drain(0)
        out_drain(1)

    return sc_kernel(yz2, idxf)


def kernel(x, W):
    B, C, N = x.shape
    O = W.shape[0]
    xt = jnp.transpose(x, (0, 2, 1))
    w1t = jnp.transpose(W[:, :C])
    wzt = jnp.transpose(W[:, C:] - W[:, :C])
    nparts = 4
    hb = B // nparts
    outs = []
    for h in range(nparts):
        xs = lax.slice_in_dim(x, h * hb, (h + 1) * hb, axis=0)
        xts = lax.slice_in_dim(xt, h * hb, (h + 1) * hb, axis=0)
        idx_g, yz = _tc_topk_proj(xts, xs, w1t, wzt)
        out2 = _sc_gather_combine(
            yz.reshape(hb * N, 2 * O),
            idx_g[:, :, :KNN].reshape(hb * N * KNN), O)
        outs.append(out2.reshape(hb, N, O))
    return jnp.concatenate(outs, axis=0).transpose(0, 2, 1)


# speedup vs baseline: 1.1301x; 1.1301x over previous
import functools

import jax
import jax.numpy as jnp
from jax import lax
from jax.experimental import pallas as pl
from jax.experimental.pallas import tpu as pltpu
from jax.experimental.pallas import tpu_sc as plsc

KNN = 20
KPAD = 32
ROW_T = 256
NEG_INF = float("-inf")


def _tc_body(xt_ref, x_ref, w1_ref, wz_ref, idx_ref, yz_ref, work_ref, *,
             n_full):
    b = pl.program_id(0)
    xt_t = xt_ref[0]
    xf = x_ref[0]

    dot = lax.dot_general(xt_t, xf, (((1,), (0,)), ((), ())),
                          preferred_element_type=jnp.float32)
    rn = jnp.sum(xt_t * xt_t, axis=1, keepdims=True)
    cn = jnp.sum(xf * xf, axis=0, keepdims=True)
    work_ref[...] = 2.0 * dot - rn - cn

    t_rows = xt_t.shape[0]
    iota_f = lax.convert_element_type(
        lax.broadcasted_iota(jnp.int32, (t_rows, n_full), 1), jnp.float32)
    slot = lax.broadcasted_iota(jnp.int32, (t_rows, KPAD), 1)

    def t_body(t, idx_acc):
        w = work_ref[...]
        m = jnp.max(w, axis=1, keepdims=True)
        cand = jnp.where(w == m, iota_f, float(n_full))
        j_f = jnp.min(cand, axis=1, keepdims=True)
        work_ref[...] = jnp.where(iota_f == j_f, NEG_INF, w)
        j = lax.convert_element_type(j_f, jnp.int32)
        return jnp.where(slot == t, j, idx_acc)

    idx_acc = lax.fori_loop(
        0, KNN, t_body, jnp.zeros((t_rows, KPAD), jnp.int32))
    idx_ref[0] = idx_acc + b * n_full

    y_t = jnp.dot(xt_t, w1_ref[...], preferred_element_type=jnp.float32)
    z_t = jnp.dot(xt_t, wz_ref[...], preferred_element_type=jnp.float32)
    yz_ref[0] = jnp.concatenate([y_t, z_t], axis=1)


def _tc_topk_proj(xt, x, w1t, wzt):
    B, N, C = xt.shape
    O = w1t.shape[1]
    nt = N // ROW_T
    return pl.pallas_call(
        functools.partial(_tc_body, n_full=N),
        grid=(B, nt),
        in_specs=[
            pl.BlockSpec((1, ROW_T, C), lambda b, i: (b, i, 0)),
            pl.BlockSpec((1, C, N), lambda b, i: (b, 0, 0)),
            pl.BlockSpec((C, O), lambda b, i: (0, 0)),
            pl.BlockSpec((C, O), lambda b, i: (0, 0)),
        ],
        out_specs=[
            pl.BlockSpec((1, ROW_T, KPAD), lambda b, i: (b, i, 0)),
            pl.BlockSpec((1, ROW_T, 2 * O), lambda b, i: (b, i, 0)),
        ],
        out_shape=[
            jax.ShapeDtypeStruct((B, N, KPAD), jnp.int32),
            jax.ShapeDtypeStruct((B, N, 2 * O), jnp.float32),
        ],
        scratch_shapes=[pltpu.VMEM((ROW_T, N), jnp.float32)],
    )(xt, x, w1t, wzt)


def _sc_gather_combine(yz2, idxf, O):
    R = yz2.shape[0]
    info = plsc.get_sparse_core_info()
    nw = info.num_cores * info.num_subcores
    rpw = R // nw
    chunk = 8
    half = chunk * KNN // 2
    nch = rpw // chunk
    nvec = O // 16

    mesh = plsc.VectorSubcoreMesh(core_axis_name="c", subcore_axis_name="s")

    @functools.partial(
        pl.kernel,
        out_type=jax.ShapeDtypeStruct((R, O), jnp.float32),
        mesh=mesh,
        scratch_types=[
            pltpu.VMEM((rpw * KNN,), jnp.int32),
            pltpu.VMEM((2, chunk * KNN, 2 * O), jnp.float32),
            pltpu.VMEM((2, chunk, 2 * O), jnp.float32),
            pltpu.VMEM((2, chunk, O), jnp.float32),
            pltpu.SemaphoreType.DMA,
            pltpu.SemaphoreType.DMA,
            pltpu.SemaphoreType.DMA,
        ],
    )
    def sc_kernel(yz_hbm, idx_hbm, out_hbm, idx_all, rows_v, z_v, acc_v,
                  gsem0, gsem1, osem):
        wid = lax.axis_index("s") * info.num_cores + lax.axis_index("c")
        base = wid * rpw
        gsems = [gsem0, gsem1]

        pltpu.sync_copy(idx_hbm.at[pl.ds(base * KNN, rpw * KNN)], idx_all)

        def issue(ci, buf):
            r0 = base + ci * chunk
            o = ci * chunk * KNN
            pltpu.async_copy(yz_hbm.at[idx_all.at[pl.ds(o, half)]],
                             rows_v.at[buf, pl.ds(0, half)], gsems[buf])
            pltpu.async_copy(yz_hbm.at[idx_all.at[pl.ds(o + half, half)]],
                             rows_v.at[buf, pl.ds(half, half)], gsems[buf])
            pltpu.async_copy(yz_hbm.at[pl.ds(r0, chunk)], z_v.at[buf],
                             gsems[buf])

        def drain(buf):
            pltpu.make_async_copy(yz_hbm.at[idx_all.at[pl.ds(0, half)]],
                                  rows_v.at[buf, pl.ds(0, half)],
                                  gsems[buf]).wait()
            pltpu.make_async_copy(yz_hbm.at[idx_all.at[pl.ds(0, half)]],
                                  rows_v.at[buf, pl.ds(half, half)],
                                  gsems[buf]).wait()
            pltpu.make_async_copy(yz_hbm.at[pl.ds(base, chunk)], z_v.at[buf],
                                  gsems[buf]).wait()

        def out_drain(buf):
            pltpu.make_async_copy(acc_v.at[buf],
                                  out_hbm.at[pl.ds(base, chunk)], osem).wait()

        def compute(ci, buf):
            r0 = base + ci * chunk
            for r in range(chunk):
                zv = [z_v[buf, r, pl.ds(O + v * 16, 16)] for v in range(nvec)]

                def k_body(k, accs, r=r, zv=zv):
                    out = []
                    for v in range(nvec):
                        g = rows_v[buf, r * KNN + k, pl.ds(v * 16, 16)]
                        s = g + zv[v]
                        out.append(accs[v] + (s - 0.8 * jnp.minimum(s, 0.0)))
                    return tuple(out)

                zero = jnp.zeros((16,), jnp.float32)
                accs = lax.fori_loop(0, KNN, k_body,
                                     tuple(zero for _ in range(nvec)))
                for v in range(nvec):
                    acc_v[buf, r, pl.ds(v * 16, 16)] = accs[v]
            pltpu.async_copy(acc_v.at[buf], out_hbm.at[pl.ds(r0, chunk)],
                             osem)

        issue(0, 0)

        def pair_body(g, _):
            c0 = 2 * g
            issue(c0 + 1, 1)
            drain(0)

            @pl.when(g > 0)
            def _():
                out_drain(0)
                out_drain(1)

            compute(c0, 0)

            @pl.when(g < nch // 2 - 1)
            def _():
                issue(c0 + 2, 0)

            drain(1)
            compute(c0 + 1, 1)
            return 0

        lax.fori_loop(0, nch // 2, pair_body, 0)
        out_drain(0)
        out_drain(1)

    return sc_kernel(yz2, idxf)


def kernel(x, W):
    B, C, N = x.shape
    O = W.shape[0]
    xt = jnp.transpose(x, (0, 2, 1))
    w1t = jnp.transpose(W[:, :C])
    wzt = jnp.transpose(W[:, C:] - W[:, :C])
    nparts = 4
    hb = B // nparts
    outs = []
    for h in range(nparts):
        xs = lax.slice_in_dim(x, h * hb, (h + 1) * hb, axis=0)
        xts = lax.slice_in_dim(xt, h * hb, (h + 1) * hb, axis=0)
        idx_g, yz = _tc_topk_proj(xts, xs, w1t, wzt)
        out2 = _sc_gather_combine(
            yz.reshape(hb * N, 2 * O),
            idx_g[:, :, :KNN].reshape(hb * N * KNN), O)
        outs.append(out2.reshape(hb, N, O))
    return jnp.concatenate(outs, axis=0).transpose(0, 2, 1)
